# Initial kernel scaffold; baseline (speedup 1.0000x reference)
import jax
import jax.numpy as jnp
from jax.experimental import pallas as pl


def kernel(x, target):
    raise NotImplementedError("write your pallas kernel here")


# speedup vs baseline: 7.9664x; 7.9664x over previous
import math

import jax
import jax.numpy as jnp
from jax.experimental import pallas as pl
from jax.experimental.pallas import tpu as pltpu

_V = 32000
_PAD = 0
_SMOOTH = 0.1
_CONF = 1.0 - _SMOOTH
_EPS = _SMOOTH / (_V - 2)
_CONST = (_V - 2) * _EPS * math.log(_EPS) + _CONF * math.log(_CONF)

_BR = 64


def _body(x_ref, t_ref, o_ref):
    step = pl.program_id(0)
    xb = x_ref[...]
    t = t_ref[...]
    cols = jax.lax.broadcasted_iota(jnp.int32, xb.shape, 1)
    srow = jnp.sum(xb, axis=1, keepdims=True)
    g = jnp.sum(jnp.where(cols == t, xb, 0.0), axis=1, keepdims=True)
    x0 = xb[:, 0:1]
    li = _CONST - _EPS * srow + _EPS * x0 + (_EPS - _CONF) * g
    partial = jnp.sum(jnp.where(t != _PAD, li, 0.0))

    @pl.when(step == 0)
    def _():
        o_ref[0, 0] = 0.0

    o_ref[0, 0] += partial


def kernel(x, target):
    n, v = x.shape
    t2 = target.astype(jnp.int32).reshape(n, 1)
    out = pl.pallas_call(
        _body,
        grid=(n // _BR,),
        in_specs=[
            pl.BlockSpec((_BR, v), lambda i: (i, 0)),
            pl.BlockSpec((_BR, 1), lambda i: (i, 0)),
        ],
        out_specs=pl.BlockSpec(memory_space=pltpu.SMEM),
        out_shape=jax.ShapeDtypeStruct((1, 1), jnp.float32),
    )(x, t2)
    return out[0, 0]
